# baseline (speedup 1.0000x reference)
import functools

import jax
import jax.numpy as jnp
import numpy as np
from jax import lax
from jax.experimental import pallas as pl
from jax.experimental.pallas import tpu as pltpu
from jax.experimental.pallas import tpu_sc as plsc

DIM = 64
MAX_POS = 8192
THETA = 10000.0
SEQ = 8192

NC = 2
NS = 16
NW = NC * NS
BPW = SEQ // NW
CHUNK = 128
NCH = BPW // CHUNK


def _fused_table():
    inv_freq = 1.0 / (THETA ** (np.arange(0, DIM, 2, dtype=np.float32) / DIM))
    t = np.arange(MAX_POS, dtype=np.float32)
    freqs = (t[:, None] * inv_freq[None, :]).astype(np.float32)
    emb = np.concatenate((freqs, freqs), axis=-1)
    return np.concatenate(
        (np.cos(emb), np.sin(emb)), axis=-1).astype(np.float32)


_TAB = _fused_table()

_mesh = plsc.VectorSubcoreMesh(core_axis_name="c", subcore_axis_name="s")


@functools.partial(
    pl.kernel,
    mesh=_mesh,
    out_type=(
        jax.ShapeDtypeStruct((1, 1, SEQ, DIM), jnp.float32),
        jax.ShapeDtypeStruct((1, 1, SEQ, DIM), jnp.float32),
    ),
    scratch_types=[
        pltpu.VMEM((BPW,), jnp.int32),
        pltpu.VMEM((BPW, 2 * DIM), jnp.float32),
        pltpu.SemaphoreType.DMA,
        pltpu.SemaphoreType.DMA,
    ],
    compiler_params=pltpu.CompilerParams(use_tc_tiling_on_sc=False),
)
def _rope_gather(tab_hbm, idx_hbm, cos_out, sin_out,
                 idx_v, rows_v, gsem, ssem):
    wid = lax.axis_index("s") * NC + lax.axis_index("c")
    base = wid * BPW
    pltpu.sync_copy(idx_hbm.at[pl.ds(base, BPW)], idx_v)
    copies = []
    for j in range(NCH):
        idx_sl = idx_v.at[pl.ds(j * CHUNK, CHUNK)]
        copies.append(pltpu.async_copy(
            tab_hbm.at[idx_sl], rows_v.at[pl.ds(j * CHUNK, CHUNK)], gsem))
    for c in copies:
        c.wait()
    outs = [
        pltpu.async_copy(
            rows_v.at[:, pl.ds(0, DIM)],
            cos_out.at[0, 0, pl.ds(base, BPW)], ssem),
        pltpu.async_copy(
            rows_v.at[:, pl.ds(DIM, DIM)],
            sin_out.at[0, 0, pl.ds(base, BPW)], ssem),
    ]
    for o in outs:
        o.wait()


def kernel(x, position_ids):
    tab = jnp.asarray(_TAB)
    idx = position_ids.reshape(SEQ).astype(jnp.int32)
    cos, sin = _rope_gather(tab, idx)
    return (cos.astype(x.dtype), sin.astype(x.dtype))


# speedup vs baseline: 1.7305x; 1.7305x over previous
import functools

import jax
import jax.numpy as jnp
import numpy as np
from jax import lax
from jax.experimental import pallas as pl
from jax.experimental.pallas import tpu as pltpu
from jax.experimental.pallas import tpu_sc as plsc

DIM = 64
MAX_POS = 8192
THETA = 10000.0
SEQ = 8192

NC = 2
NS = 16
NW = NC * NS
BPW = SEQ // NW
CHUNK = 128
NCH = BPW // CHUNK


def _fused_table():
    inv_freq = 1.0 / (THETA ** (np.arange(0, DIM, 2, dtype=np.float32) / DIM))
    t = np.arange(MAX_POS, dtype=np.float32)
    freqs = (t[:, None] * inv_freq[None, :]).astype(np.float32)
    emb = np.concatenate((freqs, freqs), axis=-1)
    return np.concatenate(
        (np.cos(emb), np.sin(emb)), axis=-1).astype(np.float32)


_TAB = _fused_table()

_mesh = plsc.VectorSubcoreMesh(core_axis_name="c", subcore_axis_name="s")


@functools.partial(
    pl.kernel,
    mesh=_mesh,
    out_type=jax.ShapeDtypeStruct((SEQ, 2 * DIM), jnp.float32),
    scratch_types=[
        pltpu.VMEM((BPW,), jnp.int32),
        pltpu.VMEM((BPW, 2 * DIM), jnp.float32),
        pltpu.SemaphoreType.DMA,
        pltpu.SemaphoreType.DMA,
    ],
)
def _rope_gather(tab_hbm, idx_hbm, wide_out, idx_v, rows_v, gsem, ssem):
    wid = lax.axis_index("s") * NC + lax.axis_index("c")
    base = wid * BPW
    pltpu.sync_copy(idx_hbm.at[pl.ds(base, BPW)], idx_v)
    copies = []
    for j in range(NCH):
        idx_sl = idx_v.at[pl.ds(j * CHUNK, CHUNK)]
        copies.append(pltpu.async_copy(
            tab_hbm.at[idx_sl], rows_v.at[pl.ds(j * CHUNK, CHUNK)], gsem))
    for c in copies:
        c.wait()
    pltpu.sync_copy(rows_v, wide_out.at[pl.ds(base, BPW)])


def kernel(x, position_ids):
    tab = jnp.asarray(_TAB)
    idx = position_ids.reshape(SEQ).astype(jnp.int32)
    wide = _rope_gather(tab, idx)
    cos = wide[:, :DIM].reshape(1, 1, SEQ, DIM).astype(x.dtype)
    sin = wide[:, DIM:].reshape(1, 1, SEQ, DIM).astype(x.dtype)
    return (cos, sin)
